# baseline (speedup 1.0000x reference)
import jax
import jax.numpy as jnp
from jax import lax
from jax.experimental import pallas as pl

_N = 10000
_E = 320000
_R = 500
_H = 8
_D = 128


def _proj_nodes(x, w):
    bn = 400

    def body(x_ref, w_ref, o_ref):
        o_ref[0] = lax.dot_general(
            x_ref[...], w_ref[0], (((1,), (1,)), ((), ())),
            preferred_element_type=jnp.float32)

    return pl.pallas_call(
        body,
        grid=(2, _N // bn),
        in_specs=[
            pl.BlockSpec((bn, _D), lambda i, j: (j, 0)),
            pl.BlockSpec((1, _H, _D), lambda i, j: (i, 0, 0)),
        ],
        out_specs=pl.BlockSpec((1, bn, _H), lambda i, j: (i, j, 0)),
        out_shape=jax.ShapeDtypeStruct((2, _N, _H), jnp.float32),
    )(x, w)


def _proj_rels(x, w):
    def body(x_ref, w_ref, o_ref):
        o_ref[...] = lax.dot_general(
            x_ref[...], w_ref[...], (((1,), (1,)), ((), ())),
            preferred_element_type=jnp.float32)

    return pl.pallas_call(
        body,
        out_shape=jax.ShapeDtypeStruct((_R, _H), jnp.float32),
    )(x, w)


def _seg_softmax(src, index, num_segments):
    m = jax.ops.segment_max(src, index, num_segments=num_segments)
    m = jnp.where(jnp.isfinite(m), m, 0.0)
    m = lax.stop_gradient(m)
    e = jnp.exp(src - m[index])
    s = jax.ops.segment_sum(e, index, num_segments=num_segments)
    return e / (s[index] + 1e-16)


def kernel(x_e, x_r, W_ah, W_at, W_ar, edge_index, rel):
    w_ht = jnp.stack([W_ah, W_at])
    pp = _proj_nodes(x_e, w_ht)
    pr = _proj_rels(x_r, W_ar)
    hi, ti = edge_index[0], edge_index[1]
    e_r = pr[rel]
    comb_h = jax.nn.leaky_relu(pp[0][hi] + e_r, negative_slope=0.01)
    comb_t = jax.nn.leaky_relu(pp[1][ti] + e_r, negative_slope=0.01)
    alpha_h = _seg_softmax(comb_h.astype(jnp.float32), hi, _N)
    alpha_t = _seg_softmax(comb_t.astype(jnp.float32), ti, _N)
    msg = x_r[rel]
    x_e_h = jax.ops.segment_sum(
        jnp.mean(alpha_h, axis=1, keepdims=True) * msg, hi, num_segments=_N)
    x_e_t = jax.ops.segment_sum(
        jnp.mean(alpha_t, axis=1, keepdims=True) * msg, ti, num_segments=_N)
    return jnp.concatenate([x_e_h, x_e_t], axis=1)
